# baseline (speedup 1.0000x reference)
import functools

import jax
import jax.numpy as jnp
from jax import lax
from jax.experimental import pallas as pl
from jax.experimental.pallas import tpu as pltpu
from jax.experimental.pallas import tpu_sc as plsc

_L = 16
_POSITIONS = 1024
_NC = 1
_NS = 16
_TILES = _NC * _NS
_PER_TILE = _POSITIONS // _TILES
_CHUNKS = _PER_TILE // _L

_GATHER_DNUMS = lax.GatherDimensionNumbers(
    offset_dims=(), collapsed_slice_dims=(0,), start_index_map=(0,))


def _make_sc_gather(vocab: int):
    mesh = plsc.VectorSubcoreMesh(core_axis_name="c", subcore_axis_name="s", num_cores=1)
    params = pltpu.CompilerParams(use_tc_tiling_on_sc=True,
                                  skip_device_barrier=True)

    @functools.partial(
        pl.kernel,
        mesh=mesh,
        out_type=jax.ShapeDtypeStruct((_POSITIONS,), jnp.float32),
        scratch_types=[
            pltpu.VMEM((8, 32), jnp.int32),
            pltpu.VMEM((_PER_TILE, 8, 128), jnp.float32),
            pltpu.VMEM((_PER_TILE,), jnp.float32),
            pltpu.SemaphoreType.DMA,
        ],
        compiler_params=params,
    )
    def sc_gather(pre_hbm, lab_hbm, out_hbm, lab_v, slot_v, res_v, sem):
        cid = lax.axis_index("c")
        sid = lax.axis_index("s")
        wid = sid * _NC + cid
        base = wid * _PER_TILE

        row0 = base >> 5
        pltpu.sync_copy(lab_hbm.at[row0 >> 3], lab_v)
        lab_vecs = []
        for c in range(_CHUNKS):
            rowsel = (row0 + (c * _L) // 32) & 7
            off = (c * _L) % 32
            lv = lab_v[0, pl.ds(off, _L)]
            for r in range(1, 8):
                lv = jnp.where(rowsel == r, lab_v[r, pl.ds(off, _L)], lv)
            lab_vecs.append(lv)

        copies = []
        for j in range(_PER_TILE):
            lab_j = lab_vecs[j // _L][j % _L]
            c16 = pl.multiple_of(lab_j & jnp.int32(~15), 16)
            g = (base >> 3) + (j >> 3)
            copies.append(pltpu.async_copy(
                pre_hbm.at[g, :, pl.ds(c16, _L)],
                slot_v.at[j, :, pl.ds(0, _L)], sem))
        for cp in copies:
            cp.wait()

        lanes = lax.iota(jnp.int32, _L)
        for c in range(_CHUNKS):
            sub_vec = lab_vecs[c] & 15
            chunk = jnp.zeros((_L,), jnp.float32)
            for k in range(_L):
                j = c * _L + k
                vrow = slot_v[j, j & 7, pl.ds(0, _L)]
                idx = jnp.zeros((_L,), jnp.int32) + sub_vec[k]
                val = lax.gather(vrow, idx[:, None], _GATHER_DNUMS, (1,),
                                 mode=lax.GatherScatterMode.PROMISE_IN_BOUNDS)
                sel = jnp.where(lanes == k, jnp.float32(1), jnp.float32(0))
                chunk = chunk + val * sel
            res_v[pl.ds(c * _L, _L)] = chunk
        pltpu.sync_copy(res_v, out_hbm.at[pl.ds(base, _PER_TILE)])

    return sc_gather


def _tc_finalize(g_ref, m_ref, out_ref):
    g = g_ref[...]
    m = m_ref[...]
    res = -jnp.sum(g * m) / jnp.sum(m)
    out_ref[...] = jnp.zeros((1, 1), jnp.float32) + res


def kernel(pre, label, mask):
    vocab = pre.shape[2]
    pre3 = pre.reshape(128, 8, vocab)
    lab3 = label.astype(jnp.int32).reshape(4, 8, 32)

    gathered = _make_sc_gather(vocab)(pre3, lab3)

    out = pl.pallas_call(
        _tc_finalize,
        out_shape=jax.ShapeDtypeStruct((1, 1), jnp.float32),
    )(gathered.reshape(8, 128), mask.astype(jnp.float32).reshape(8, 128))
    return out[0, 0]


# speedup vs baseline: 1.0237x; 1.0237x over previous
import functools

import jax
import jax.numpy as jnp
from jax import lax
from jax.experimental import pallas as pl
from jax.experimental.pallas import tpu as pltpu
from jax.experimental.pallas import tpu_sc as plsc

_L = 16
_POSITIONS = 1024
_NC = 2
_NS = 16
_TILES = _NC * _NS
_PER_TILE = _POSITIONS // _TILES
_CHUNKS = _PER_TILE // _L

_GATHER_DNUMS = lax.GatherDimensionNumbers(
    offset_dims=(), collapsed_slice_dims=(0,), start_index_map=(0,))


def _make_sc_gather(vocab: int):
    mesh = plsc.VectorSubcoreMesh(core_axis_name="c", subcore_axis_name="s")
    params = pltpu.CompilerParams(use_tc_tiling_on_sc=True,
                                  skip_device_barrier=True,
                                  needs_layout_passes=False)

    @functools.partial(
        pl.kernel,
        mesh=mesh,
        out_type=jax.ShapeDtypeStruct((_POSITIONS,), jnp.float32),
        scratch_types=[
            pltpu.VMEM((_PER_TILE,), jnp.int32),
            pltpu.VMEM((_PER_TILE, 8, 128), jnp.float32),
            pltpu.VMEM((_PER_TILE,), jnp.float32),
            pltpu.SemaphoreType.DMA,
        ],
        compiler_params=params,
    )
    def sc_gather(pre_hbm, lab_hbm, out_hbm, lab_v, slot_v, res_v, sem):
        cid = lax.axis_index("c")
        sid = lax.axis_index("s")
        wid = sid * _NC + cid
        base = wid * _PER_TILE

        pltpu.sync_copy(lab_hbm.at[pl.ds(base, _PER_TILE)], lab_v)
        lab_vecs = [lab_v[pl.ds(c * _L, _L)] for c in range(_CHUNKS)]

        copies = []
        for j in range(_PER_TILE):
            lab_j = lab_vecs[j // _L][j % _L]
            c16 = pl.multiple_of(lab_j & jnp.int32(~15), 16)
            g = (base >> 3) + (j >> 3)
            copies.append(pltpu.async_copy(
                pre_hbm.at[g, :, pl.ds(c16, _L)],
                slot_v.at[j, :, pl.ds(0, _L)], sem))
        for cp in copies:
            cp.wait()

        lanes = lax.iota(jnp.int32, _L)
        for c in range(_CHUNKS):
            sub_vec = lab_vecs[c] & 15
            j_vec = lanes + c * _L
            r_vec = lanes & 7
            res_v[pl.ds(c * _L, _L)] = plsc.load_gather(
                slot_v, [j_vec, r_vec, sub_vec])
        pltpu.sync_copy(res_v, out_hbm.at[pl.ds(base, _PER_TILE)])

    return sc_gather


def _tc_finalize(g_ref, m_ref, out_ref):
    g = g_ref[...]
    m = m_ref[...]
    res = -jnp.sum(g * m) / jnp.sum(m)
    out_ref[...] = jnp.zeros((1, 1), jnp.float32) + res


def kernel(pre, label, mask):
    vocab = pre.shape[2]
    pre3 = pre.reshape(128, 8, vocab)
    lab = label.reshape(-1).astype(jnp.int32)

    gathered = _make_sc_gather(vocab)(pre3, lab)

    out = pl.pallas_call(
        _tc_finalize,
        out_shape=jax.ShapeDtypeStruct((1, 1), jnp.float32),
    )(gathered.reshape(8, 128), mask.astype(jnp.float32).reshape(8, 128))
    return out[0, 0]
